# Initial kernel scaffold; baseline (speedup 1.0000x reference)
import jax
import jax.numpy as jnp
from jax.experimental import pallas as pl


def kernel(pcd, divide_ratio, params, conv_w, conv_b):
    raise NotImplementedError("write your pallas kernel here")


# speedup vs baseline: 5.9972x; 5.9972x over previous
import functools

import jax
import jax.numpy as jnp
from jax import lax
from jax.experimental import pallas as pl
from jax.experimental.pallas import tpu as pltpu
from jax.experimental.pallas import tpu_sc as plsc


def _fps(x, y, z, npoint):
    B, N = x.shape

    def body(x_ref, y_ref, z_ref, ox_ref, oy_ref, oz_ref):
        xv = x_ref[...]
        yv = y_ref[...]
        zv = z_ref[...]
        iota = lax.broadcasted_iota(jnp.int32, (B, N), 1)
        iota_s = lax.broadcasted_iota(jnp.int32, (B, npoint), 1)

        def step(i, carry):
            dist, far, ox, oy, oz = carry
            m = iota == far
            cx = jnp.sum(jnp.where(m, xv, 0.0), axis=1, keepdims=True)
            cy = jnp.sum(jnp.where(m, yv, 0.0), axis=1, keepdims=True)
            cz = jnp.sum(jnp.where(m, zv, 0.0), axis=1, keepdims=True)
            dx = xv - cx
            dy = yv - cy
            dz = zv - cz
            d = (dx * dx + dy * dy) + dz * dz
            dist = jnp.minimum(dist, d)
            maxv = jnp.max(dist, axis=1, keepdims=True)
            far2 = jnp.min(jnp.where(dist == maxv, iota, N), axis=1,
                           keepdims=True)
            sel = iota_s == i
            ox = jnp.where(sel, cx, ox)
            oy = jnp.where(sel, cy, oy)
            oz = jnp.where(sel, cz, oz)
            return dist, far2, ox, oy, oz

        init = (jnp.full((B, N), 1e10, jnp.float32),
                jnp.zeros((B, 1), jnp.int32),
                jnp.zeros((B, npoint), jnp.float32),
                jnp.zeros((B, npoint), jnp.float32),
                jnp.zeros((B, npoint), jnp.float32))
        _, _, ox, oy, oz = lax.fori_loop(0, npoint, step, init)
        ox_ref[...] = ox
        oy_ref[...] = oy
        oz_ref[...] = oz

    f32 = jnp.float32
    out = pl.pallas_call(
        body,
        out_shape=[jax.ShapeDtypeStruct((B, npoint), f32)] * 3,
    )(x, y, z)
    return out


def _select(xyzT, new_xyz, radius, ns, sc):
    B, _, N = xyzT.shape
    S = new_xyz.shape[1]
    NB = N // 16
    r2 = radius * radius

    def body(xt_ref, nx_ref, out_ref):
        b = pl.program_id(0)
        xt = xt_ref[0]
        C = nx_ref[0]
        px = xt[0:1, :]
        py = xt[1:2, :]
        pz = xt[2:3, :]
        d2 = (px * px + py * py) + pz * pz
        cx = C[:, 0:1]
        cy = C[:, 1:2]
        cz = C[:, 2:3]
        cs2 = (cx * cx + cy * cy) + cz * cz
        dotp = lax.dot_general(C, xt, (((1,), (0,)), ((), ())),
                               preferred_element_type=jnp.float32)
        sqr = (cs2 + d2) - 2.0 * dotp
        maskf = (sqr <= r2).astype(jnp.float32)

        rown = lax.broadcasted_iota(jnp.int32, (N, NB), 0)
        colj = lax.broadcasted_iota(jnp.int32, (N, NB), 1)
        inb = (rown // 16) == colj
        bmat = inb.astype(jnp.float32)
        pow2 = (1 << (rown % 16)).astype(jnp.float32)
        b2 = jnp.where(inb, pow2, 0.0)
        cnt = lax.dot_general(maskf, bmat, (((1,), (0,)), ((), ())),
                              preferred_element_type=jnp.float32)
        wrd = lax.dot_general(maskf, b2, (((1,), (0,)), ((), ())),
                              preferred_element_type=jnp.float32)

        inc = cnt
        sh = 1
        while sh < NB:
            zpad = jnp.zeros((sc, sh), jnp.float32)
            inc = inc + jnp.concatenate([zpad, inc[:, :NB - sh]], axis=1)
            sh *= 2
        base = inc - cnt
        total = inc[:, NB - 1:NB]

        basex = base[:, None, :]
        kf = lax.broadcasted_iota(jnp.int32, (sc, ns, 1), 1).astype(jnp.float32)
        cntk = jnp.sum((basex <= kf).astype(jnp.float32), axis=2,
                       keepdims=True)
        bk = cntk - 1.0
        jj = lax.broadcasted_iota(jnp.int32, (sc, ns, NB), 2).astype(jnp.float32)
        oh = (jj == bk).astype(jnp.float32)
        base_at = jnp.sum(oh * basex, axis=2)
        wrd_at = jnp.sum(oh * wrd[:, None, :], axis=2)

        w = wrd_at.astype(jnp.int32)
        r = (kf[:, :, 0] - base_at).astype(jnp.int32)
        run = jnp.zeros((sc, ns), jnp.int32)
        nloc = jnp.zeros((sc, ns), jnp.int32)
        for t in range(16):
            bit = (w >> t) & 1
            hit = (bit == 1) & (run == r)
            nloc = jnp.where(hit, t, nloc)
            run = run + bit
        n_idx = bk[:, :, 0].astype(jnp.int32) * 16 + nloc

        k_i = lax.broadcasted_iota(jnp.int32, (sc, ns), 1)
        tot_i = total.astype(jnp.int32)
        idx = jnp.where(k_i < tot_i, n_idx,
                        jnp.broadcast_to(n_idx[:, 0:1], (sc, ns)))
        out_ref[0] = idx + b * N

    grid = (B, S // sc)
    return pl.pallas_call(
        body,
        grid=grid,
        in_specs=[
            pl.BlockSpec((1, 3, N), lambda b, c: (b, 0, 0)),
            pl.BlockSpec((1, sc, 3), lambda b, c: (b, c, 0)),
        ],
        out_specs=pl.BlockSpec((1, sc, ns), lambda b, c: (b, c, 0)),
        out_shape=jax.ShapeDtypeStruct((B, S, ns), jnp.int32),
    )(xyzT, new_xyz)


def _gather_rows(table, idx_all):
    T = idx_all.shape[0]
    info = plsc.get_sparse_core_info()
    NC, NS = info.num_cores, info.num_subcores
    NW = NC * NS
    per_w = T // NW
    CH = 128
    n_ch = per_w // CH
    mesh = plsc.VectorSubcoreMesh(core_axis_name="c", subcore_axis_name="s")

    @functools.partial(
        pl.kernel,
        out_type=jax.ShapeDtypeStruct((T, 16), jnp.float32),
        mesh=mesh,
        compiler_params=pltpu.CompilerParams(use_tc_tiling_on_sc=False),
        scratch_types=[
            pltpu.VMEM((CH,), jnp.int32),
            pltpu.VMEM((CH, 16), jnp.float32),
            pltpu.SemaphoreType.DMA,
        ],
    )
    def k(table_hbm, idx_hbm, out_hbm, idx_v, rows_v, sem):
        wid = lax.axis_index("s") * NC + lax.axis_index("c")
        wbase = wid * per_w

        def chunk(g, carry):
            off = wbase + g * CH
            pltpu.sync_copy(idx_hbm.at[pl.ds(off, CH)], idx_v)
            pltpu.async_copy(table_hbm.at[idx_v], rows_v, sem).wait()
            pltpu.sync_copy(rows_v, out_hbm.at[pl.ds(off, CH)])
            return carry

        lax.fori_loop(0, n_ch, chunk, 0)

    return k(table, idx_all)


def _mlp_scale(rows_s, new_xyz, layers, cw_slice, ns, sc):
    B, S = new_xyz.shape[0], new_xyz.shape[1]
    (W1, b1), (W2, b2), (W3, b3) = layers
    C1, C2, C3 = W1.shape[1], W2.shape[1], W3.shape[1]
    W1p = jnp.pad(W1, ((0, 13), (0, 0)))

    def body(rows_ref, nx_ref, w1_ref, b1_ref, w2_ref, b2_ref, w3_ref, b3_ref,
             cw_ref, out_ref):
        g = rows_ref[0]
        C = nx_ref[0]
        dot = lambda a, b: lax.dot_general(
            a, b, (((1,), (0,)), ((), ())), preferred_element_type=jnp.float32)
        cW = dot(C, w1_ref[0:3, :])
        h = dot(g, w1_ref[...])
        h = h.reshape(sc, ns, C1) - cW[:, None, :] + b1_ref[...]
        h = jnp.maximum(h, 0.0)
        h = dot(h.reshape(sc * ns, C1), w2_ref[...]) + b2_ref[...]
        h = jnp.maximum(h, 0.0)
        h = dot(h, w3_ref[...]) + b3_ref[...]
        h = jnp.maximum(h, 0.0)
        f = jnp.max(h.reshape(sc, ns, C3), axis=1)
        out_ref[0] = dot(f, cw_ref[...])

    grid = (B, S // sc)
    full = lambda shape: pl.BlockSpec(shape, lambda b, c: tuple(0 for _ in shape))
    return pl.pallas_call(
        body,
        grid=grid,
        in_specs=[
            pl.BlockSpec((1, sc * ns, 16), lambda b, c: (b, c, 0)),
            pl.BlockSpec((1, sc, 3), lambda b, c: (b, c, 0)),
            full((16, C1)),
            full((1, C1)),
            full((C1, C2)),
            full((1, C2)),
            full((C2, C3)),
            full((1, C3)),
            full((C3, 1)),
        ],
        out_specs=pl.BlockSpec((1, sc, 1), lambda b, c: (b, c, 0)),
        out_shape=jax.ShapeDtypeStruct((B, S, 1), jnp.float32),
    )(rows_s, new_xyz, W1p, b1.reshape(1, C1), W2, b2.reshape(1, C2),
      W3, b3.reshape(1, C3), cw_slice)


def kernel(pcd, divide_ratio, params, conv_w, conv_b):
    B, N, _ = pcd.shape
    S = N // 8
    radii = (0.1, 0.2, 0.4)
    nsamples = (16, 32, 128)
    sel_sc = (128, 64, 32)
    mlp_sc = (64, 64, 64)

    x = pcd[:, :, 0]
    y = pcd[:, :, 1]
    z = pcd[:, :, 2]
    nx, ny, nz = _fps(x, y, z, S)
    new_xyz = jnp.stack([nx, ny, nz], axis=-1)
    xyzT = jnp.transpose(pcd, (0, 2, 1))

    idxs = [
        _select(xyzT, new_xyz, radii[s], nsamples[s], sel_sc[s])
        for s in range(3)
    ]
    idx_all = jnp.concatenate([i.reshape(-1) for i in idxs])
    table = jnp.pad(pcd.reshape(B * N, 3), ((0, 0), (0, 13)))
    rows = _gather_rows(table, idx_all)

    off = 0
    coff = 0
    acc = None
    for s in range(3):
        ns = nsamples[s]
        C3 = params[s][2][0].shape[1]
        rows_s = rows[off:off + B * S * ns].reshape(B, S * ns, 16)
        off += B * S * ns
        cw_slice = conv_w[coff:coff + C3]
        coff += C3
        p = _mlp_scale(rows_s, new_xyz, params[s], cw_slice, ns, mlp_sc[s])
        acc = p if acc is None else acc + p
    return acc + conv_b
